# baseline (speedup 1.0000x reference)
import jax
import jax.numpy as jnp
from jax import lax
from jax.experimental import pallas as pl
from jax.experimental.pallas import tpu as pltpu
from jax.experimental.pallas import tpu_sc as plsc

VOCAB = 1000000
EMB = 32
BATCH = 4096
SEQ = 200

TOTAL = BATCH * SEQ
NUM_WORKERS = 32
PER_WORKER = TOTAL // NUM_WORKERS
STREAM = 128
K = 10
CHUNK = STREAM * K
NUM_CHUNKS = PER_WORKER // CHUNK
LANES = 16


def _emb_body(x_hbm, table_hbm, out_hbm, mask_hbm,
              idx0, idx1, rows0, rows1, mask0, mask1, sem0, sem1):
    idx = (idx0, idx1)
    rows = (rows0, rows1)
    maskv = (mask0, mask1)
    sems = (sem0, sem1)
    wid = lax.axis_index("s") * 2 + lax.axis_index("c")
    wbase = wid * PER_WORKER

    def stage_fire(g, b):
        base = wbase + g * CHUNK
        pltpu.sync_copy(x_hbm.at[pl.ds(base, CHUNK)], idx[b])
        pltpu.async_copy(
            table_hbm.at[pl.ds(base, CHUNK)], rows[b], sems[b])

    def drain_write(g, b):
        base = wbase + g * CHUNK

        def mask_body(i, c):
            v = idx[b][pl.ds(i * LANES, LANES)]
            maskv[b][pl.ds(i * LANES, LANES)] = jnp.where(
                v != 0, jnp.full((LANES,), 1.0, jnp.float32),
                jnp.full((LANES,), 0.0, jnp.float32))
            return c

        lax.fori_loop(0, 1, mask_body, 0)
        pltpu.make_async_copy(
            table_hbm.at[pl.ds(0, CHUNK)], rows[b], sems[b]).wait()
        pltpu.sync_copy(maskv[b], mask_hbm.at[pl.ds(base, CHUNK)])

    stage_fire(0, 0)

    @pl.loop(0, NUM_CHUNKS - 2, step=2)
    def _(t):
        stage_fire(t + 1, 1)
        drain_write(t, 0)
        stage_fire(t + 2, 0)
        drain_write(t + 1, 1)

    stage_fire(NUM_CHUNKS - 1, 1)
    drain_write(NUM_CHUNKS - 2, 0)
    drain_write(NUM_CHUNKS - 1, 1)


@jax.jit
def _emb_call(xf, table):
    mesh = plsc.VectorSubcoreMesh(core_axis_name="c", subcore_axis_name="s")
    fn = pl.kernel(
        _emb_body,
        out_type=[
            jax.ShapeDtypeStruct((TOTAL, EMB), jnp.float32),
            jax.ShapeDtypeStruct((TOTAL,), jnp.float32),
        ],
        mesh=mesh,
        scratch_types=[
            pltpu.VMEM((CHUNK,), jnp.int32),
            pltpu.VMEM((CHUNK,), jnp.int32),
            pltpu.VMEM((CHUNK, EMB), jnp.float32),
            pltpu.VMEM((CHUNK, EMB), jnp.float32),
            pltpu.VMEM((CHUNK,), jnp.float32),
            pltpu.VMEM((CHUNK,), jnp.float32),
            pltpu.SemaphoreType.DMA,
            pltpu.SemaphoreType.DMA,
        ],
        compiler_params=pltpu.CompilerParams(use_tc_tiling_on_sc=False),
    )
    return fn(xf, table)


def kernel(x, table):
    xf = x.reshape(TOTAL).astype(jnp.int32)
    emb_flat, mask_flat = _emb_call(xf, table)
    return (emb_flat.reshape(BATCH, SEQ, EMB), mask_flat.reshape(BATCH, SEQ))


# speedup vs baseline: 1.0145x; 1.0145x over previous
import jax
import jax.numpy as jnp
from jax import lax
from jax.experimental import pallas as pl
from jax.experimental.pallas import tpu as pltpu
from jax.experimental.pallas import tpu_sc as plsc

VOCAB = 1000000
EMB = 32
BATCH = 4096
SEQ = 200

TOTAL = BATCH * SEQ
NUM_WORKERS = 32
PER_WORKER = TOTAL // NUM_WORKERS
STREAM = 128
K = 10
CHUNK = STREAM * K
NUM_CHUNKS = PER_WORKER // CHUNK
LANES = 16


def _emb_body(x_hbm, table_hbm, out_hbm, mask_hbm,
              idx0, idx1, rows0, rows1, mask0, mask1, sem0, sem1):
    idx = (idx0, idx1)
    rows = (rows0, rows1)
    maskv = (mask0, mask1)
    sems = (sem0, sem1)
    wid = lax.axis_index("s") * 2 + lax.axis_index("c")
    wbase = wid * PER_WORKER

    def stage_fire(g, b):
        base = wbase + g * CHUNK
        pltpu.sync_copy(x_hbm.at[pl.ds(base, CHUNK)], idx[b])
        for j in range(K):
            pltpu.async_copy(
                table_hbm.at[idx[b].at[pl.ds(j * STREAM, STREAM)]],
                rows[b].at[pl.ds(j * STREAM, STREAM)],
                sems[b],
            )

    def drain_write(g, b):
        base = wbase + g * CHUNK

        def mask_body(i, c):
            v = idx[b][pl.ds(i * LANES, LANES)]
            maskv[b][pl.ds(i * LANES, LANES)] = jnp.where(
                v != 0, jnp.full((LANES,), 1.0, jnp.float32),
                jnp.full((LANES,), 0.0, jnp.float32))
            return c

        lax.fori_loop(0, CHUNK // LANES, mask_body, 0)
        pltpu.make_async_copy(
            table_hbm.at[pl.ds(0, CHUNK)], rows[b], sems[b]).wait()
        pltpu.sync_copy(rows[b], out_hbm.at[pl.ds(base, CHUNK)])
        pltpu.sync_copy(maskv[b], mask_hbm.at[pl.ds(base, CHUNK)])

    stage_fire(0, 0)

    @pl.loop(0, NUM_CHUNKS - 2, step=2)
    def _(t):
        stage_fire(t + 1, 1)
        drain_write(t, 0)
        stage_fire(t + 2, 0)
        drain_write(t + 1, 1)

    stage_fire(NUM_CHUNKS - 1, 1)
    drain_write(NUM_CHUNKS - 2, 0)
    drain_write(NUM_CHUNKS - 1, 1)


def _emb_pallas(xf, table):
    mesh = plsc.VectorSubcoreMesh(core_axis_name="c", subcore_axis_name="s")
    fn = pl.kernel(
        _emb_body,
        out_type=[
            jax.ShapeDtypeStruct((TOTAL, EMB), jnp.float32),
            jax.ShapeDtypeStruct((TOTAL,), jnp.float32),
        ],
        mesh=mesh,
        scratch_types=[
            pltpu.VMEM((CHUNK,), jnp.int32),
            pltpu.VMEM((CHUNK,), jnp.int32),
            pltpu.VMEM((CHUNK, EMB), jnp.float32),
            pltpu.VMEM((CHUNK, EMB), jnp.float32),
            pltpu.VMEM((CHUNK,), jnp.float32),
            pltpu.VMEM((CHUNK,), jnp.float32),
            pltpu.SemaphoreType.DMA,
            pltpu.SemaphoreType.DMA,
        ],
        compiler_params=pltpu.CompilerParams(use_tc_tiling_on_sc=False),
    )
    return fn(xf, table)


@jax.jit
def _impl(x, table):
    xf = x.T.reshape(TOTAL).astype(jnp.int32)
    emb_flat, mask_flat = _emb_pallas(xf, table)
    emb = emb_flat.reshape(SEQ, BATCH, EMB).transpose(1, 0, 2)
    mask = mask_flat.reshape(SEQ, BATCH).T
    return (emb, mask)


def kernel(x, table):
    return _impl(x, table)
